# baseline (speedup 1.0000x reference)
import functools

import jax
import jax.numpy as jnp
from jax import lax
from jax.experimental import pallas as pl
from jax.experimental.pallas import tpu as pltpu
from jax.experimental.pallas import tpu_sc as plsc

V = 1000
N_TOKENS = 16
HID = 256
B, S, L = 8, 4, 256
NW = 32
TPW = (B * S * L) // NW
ROWS_PER_SLAB = N_TOKENS + L
CHUNK = 32
MAIN_W = 896
TAIL_BLK = 7


def _tc_prep(wte, u, v, shared_prompt, mlp_w, mlp_b):
    RB = 200

    def body(wte_ref, u_ref, v_ref, sp_ref, w_ref, b_ref,
             main_ref, tail_ref, learned_ref):
        main_ref[...] = wte_ref[:, :MAIN_W]
        tail_ref[...] = jnp.concatenate(
            [wte_ref[:, MAIN_W:], jnp.zeros((RB, 128 - (V - MAIN_W)),
                                            jnp.float32)], axis=1)
        learned = (u_ref[...] * v_ref[...]) * sp_ref[...]
        learned_ref[...] = (
            jnp.dot(learned, w_ref[...], preferred_element_type=jnp.float32)
            + b_ref[...][None, :]
        )

    grid_spec = pltpu.PrefetchScalarGridSpec(
        num_scalar_prefetch=0,
        grid=(V // RB,),
        in_specs=[
            pl.BlockSpec((RB, V), lambda i: (i, 0)),
            pl.BlockSpec((N_TOKENS, 1), lambda i: (0, 0)),
            pl.BlockSpec((1, HID), lambda i: (0, 0)),
            pl.BlockSpec((N_TOKENS, HID), lambda i: (0, 0)),
            pl.BlockSpec((HID, V), lambda i: (0, 0)),
            pl.BlockSpec((V,), lambda i: (0,)),
        ],
        out_specs=[
            pl.BlockSpec((RB, MAIN_W), lambda i: (i, 0)),
            pl.BlockSpec((RB, 128), lambda i: (i, 0)),
            pl.BlockSpec((N_TOKENS, V), lambda i: (0, 0)),
        ],
    )
    return pl.pallas_call(
        body,
        grid_spec=grid_spec,
        out_shape=[
            jax.ShapeDtypeStruct((V, MAIN_W), jnp.float32),
            jax.ShapeDtypeStruct((V, 128), jnp.float32),
            jax.ShapeDtypeStruct((N_TOKENS, V), jnp.float32),
        ],
    )(wte, u, v, shared_prompt, mlp_w, mlp_b)


def _sc_main(tokens_flat, wte_main, wte_tail, learned):
    mesh = plsc.VectorSubcoreMesh(core_axis_name="c", subcore_axis_name="s")
    n_chunks = TPW // CHUNK

    @functools.partial(
        pl.kernel,
        out_type=(
            jax.ShapeDtypeStruct((B * S * ROWS_PER_SLAB, V), jnp.float32),
            jax.ShapeDtypeStruct((B * S * L, 128), jnp.float32),
        ),
        mesh=mesh,
        scratch_types=[
            pltpu.VMEM((n_chunks, CHUNK), jnp.int32),
            pltpu.VMEM((CHUNK, MAIN_W), jnp.float32),
            pltpu.VMEM((CHUNK, MAIN_W), jnp.float32),
            pltpu.VMEM((CHUNK, MAIN_W), jnp.float32),
            pltpu.VMEM((CHUNK, 128), jnp.float32),
            pltpu.VMEM((CHUNK, 128), jnp.float32),
            pltpu.VMEM((CHUNK, 128), jnp.float32),
            pltpu.VMEM((N_TOKENS, V), jnp.float32),
            pltpu.SemaphoreType.DMA,
            pltpu.SemaphoreType.DMA,
            pltpu.SemaphoreType.DMA,
            pltpu.SemaphoreType.DMA,
            pltpu.SemaphoreType.DMA,
            pltpu.SemaphoreType.DMA,
        ],
    )
    def k(tok_hbm, wmain_hbm, wtail_hbm, learned_hbm, out_hbm, tail_hbm,
          idx_v, m0, m1, m2, t0, t1, t2, learned_v,
          gs0, gs1, gs2, ss0, ss1, ss2):
        wid = lax.axis_index("s") * 2 + lax.axis_index("c")
        out_base = wid * ROWS_PER_SLAB
        tail_base = wid * TPW

        pltpu.sync_copy(tok_hbm.at[wid], idx_v)

        mbufs = (m0, m1, m2)
        tbufs = (t0, t1, t2)
        gsems = (gs0, gs1, gs2)
        ssems = (ss0, ss1, ss2)

        def gather(c):
            return (
                pltpu.async_copy(wmain_hbm.at[idx_v.at[c]], mbufs[c % 3],
                                 gsems[c % 3]),
                pltpu.async_copy(wtail_hbm.at[idx_v.at[c]], tbufs[c % 3],
                                 gsems[c % 3]),
            )

        def scatter(c):
            return (
                pltpu.async_copy(
                    mbufs[c % 3],
                    out_hbm.at[pl.ds(out_base + N_TOKENS + c * CHUNK, CHUNK),
                               pl.ds(0, MAIN_W)],
                    ssems[c % 3],
                ),
                pltpu.async_copy(
                    tbufs[c % 3],
                    tail_hbm.at[pl.ds(tail_base + c * CHUNK, CHUNK)],
                    ssems[c % 3],
                ),
            )

        gathers = {0: gather(0), 1: gather(1)}

        pltpu.sync_copy(learned_hbm, learned_v)
        pltpu.sync_copy(learned_v, out_hbm.at[pl.ds(out_base, N_TOKENS)])

        scatters = {}
        for c in range(n_chunks):
            for h in gathers[c]:
                h.wait()
            scatters[c] = scatter(c)
            nxt = c + 2
            if nxt < n_chunks:
                if nxt - 3 >= 0:
                    for h in scatters[nxt - 3]:
                        h.wait()
                gathers[nxt] = gather(nxt)
        for c in range(max(0, n_chunks - 3), n_chunks):
            for h in scatters[c]:
                h.wait()

    return k(tokens_flat, wte_main, wte_tail, learned)


def _tc_tail(out, tail, learned):
    SLABS = 8

    def body(_, tail_ref, learned_ref, out_ref):
        for s in range(SLABS):
            out_ref[s * ROWS_PER_SLAB:s * ROWS_PER_SLAB + N_TOKENS, :] = (
                learned_ref[...]
            )
            out_ref[s * ROWS_PER_SLAB + N_TOKENS:(s + 1) * ROWS_PER_SLAB,
                    :] = tail_ref[s * L:(s + 1) * L, :]

    grid_spec = pltpu.PrefetchScalarGridSpec(
        num_scalar_prefetch=0,
        grid=(NW // SLABS,),
        in_specs=[
            pl.BlockSpec(memory_space=pl.ANY),
            pl.BlockSpec((SLABS * L, 128), lambda i: (i, 0)),
            pl.BlockSpec((N_TOKENS, 128), lambda i: (0, TAIL_BLK)),
        ],
        out_specs=pl.BlockSpec((SLABS * ROWS_PER_SLAB, 128),
                               lambda i: (i, TAIL_BLK)),
    )
    return pl.pallas_call(
        body,
        grid_spec=grid_spec,
        out_shape=jax.ShapeDtypeStruct((B * S * ROWS_PER_SLAB, V),
                                       jnp.float32),
        input_output_aliases={0: 0},
    )(out, tail, learned)


def kernel(tokens, wte, mlp_w, mlp_b, shared_prompt, u, v):
    wte_main, wte_tail, learned = _tc_prep(wte, u, v, shared_prompt,
                                           mlp_w, mlp_b)
    tokens_flat = tokens.reshape(NW, TPW // CHUNK, CHUNK).astype(jnp.int32)
    out, tail = _sc_main(tokens_flat, wte_main, wte_tail, learned)
    out = _tc_tail(out, tail, learned)
    return out.reshape(B, S, ROWS_PER_SLAB, V)


# speedup vs baseline: 1.0374x; 1.0374x over previous
import functools

import jax
import jax.numpy as jnp
from jax import lax
from jax.experimental import pallas as pl
from jax.experimental.pallas import tpu as pltpu
from jax.experimental.pallas import tpu_sc as plsc

V = 1000
N_TOKENS = 16
HID = 256
B, S, L = 8, 4, 256
NW = 32
TPW = (B * S * L) // NW
ROWS_PER_SLAB = N_TOKENS + L
CHUNK = 32
MAIN_W = 896
TAIL_BLK = 7


def _learned_prompt(u, v, shared_prompt, mlp_w, mlp_b):
    def body(u_ref, v_ref, sp_ref, w_ref, b_ref, out_ref):
        learned = (u_ref[...] * v_ref[...]) * sp_ref[...]
        out_ref[...] = (
            jnp.dot(learned, w_ref[...], preferred_element_type=jnp.float32)
            + b_ref[...][None, :]
        )

    return pl.pallas_call(
        body,
        out_shape=jax.ShapeDtypeStruct((N_TOKENS, V), jnp.float32),
    )(u, v, shared_prompt, mlp_w, mlp_b)


def _sc_main(tokens_flat, wte_main, wte_tail, learned):
    mesh = plsc.VectorSubcoreMesh(core_axis_name="c", subcore_axis_name="s")
    n_chunks = TPW // CHUNK

    @functools.partial(
        pl.kernel,
        out_type=(
            jax.ShapeDtypeStruct((B * S * ROWS_PER_SLAB, V), jnp.float32),
            jax.ShapeDtypeStruct((B * S * L, 128), jnp.float32),
        ),
        mesh=mesh,
        scratch_types=[
            pltpu.VMEM((n_chunks, CHUNK), jnp.int32),
            pltpu.VMEM((CHUNK, MAIN_W), jnp.float32),
            pltpu.VMEM((CHUNK, MAIN_W), jnp.float32),
            pltpu.VMEM((CHUNK, MAIN_W), jnp.float32),
            pltpu.VMEM((CHUNK, 128), jnp.float32),
            pltpu.VMEM((CHUNK, 128), jnp.float32),
            pltpu.VMEM((CHUNK, 128), jnp.float32),
            pltpu.VMEM((N_TOKENS, V), jnp.float32),
            pltpu.SemaphoreType.DMA,
            pltpu.SemaphoreType.DMA,
            pltpu.SemaphoreType.DMA,
            pltpu.SemaphoreType.DMA,
            pltpu.SemaphoreType.DMA,
            pltpu.SemaphoreType.DMA,
        ],
    )
    def k(tok_hbm, wmain_hbm, wtail_hbm, learned_hbm, out_hbm, tail_hbm,
          idx_v, m0, m1, m2, t0, t1, t2, learned_v,
          gs0, gs1, gs2, ss0, ss1, ss2):
        wid = lax.axis_index("s") * 2 + lax.axis_index("c")
        out_base = wid * ROWS_PER_SLAB
        tail_base = wid * TPW

        pltpu.sync_copy(tok_hbm.at[wid], idx_v)

        mbufs = (m0, m1, m2)
        tbufs = (t0, t1, t2)
        gsems = (gs0, gs1, gs2)
        ssems = (ss0, ss1, ss2)

        def gather(c):
            return (
                pltpu.async_copy(wmain_hbm.at[idx_v.at[c]], mbufs[c % 3],
                                 gsems[c % 3]),
                pltpu.async_copy(wtail_hbm.at[idx_v.at[c]], tbufs[c % 3],
                                 gsems[c % 3]),
            )

        def scatter(c):
            return (
                pltpu.async_copy(
                    mbufs[c % 3],
                    out_hbm.at[pl.ds(out_base + N_TOKENS + c * CHUNK, CHUNK),
                               pl.ds(0, MAIN_W)],
                    ssems[c % 3],
                ),
                pltpu.async_copy(
                    tbufs[c % 3],
                    tail_hbm.at[pl.ds(tail_base + c * CHUNK, CHUNK)],
                    ssems[c % 3],
                ),
            )

        gathers = {0: gather(0), 1: gather(1)}

        pltpu.sync_copy(learned_hbm, learned_v)
        pltpu.sync_copy(learned_v, out_hbm.at[pl.ds(out_base, N_TOKENS)])

        scatters = {}
        for c in range(n_chunks):
            for h in gathers[c]:
                h.wait()
            scatters[c] = scatter(c)
            nxt = c + 2
            if nxt < n_chunks:
                if nxt - 3 >= 0:
                    for h in scatters[nxt - 3]:
                        h.wait()
                gathers[nxt] = gather(nxt)
        for c in range(max(0, n_chunks - 3), n_chunks):
            for h in scatters[c]:
                h.wait()

    return k(tokens_flat, wte_main, wte_tail, learned)


def _tc_tail(out, tail, learned):
    SLABS = 8

    def body(_, tail_ref, learned_ref, out_ref):
        for s in range(SLABS):
            out_ref[s * ROWS_PER_SLAB:s * ROWS_PER_SLAB + N_TOKENS, :] = (
                learned_ref[...]
            )
            out_ref[s * ROWS_PER_SLAB + N_TOKENS:(s + 1) * ROWS_PER_SLAB,
                    :] = tail_ref[s * L:(s + 1) * L, :]

    grid_spec = pltpu.PrefetchScalarGridSpec(
        num_scalar_prefetch=0,
        grid=(NW // SLABS,),
        in_specs=[
            pl.BlockSpec(memory_space=pl.ANY),
            pl.BlockSpec((SLABS * L, 128), lambda i: (i, 0)),
            pl.BlockSpec((N_TOKENS, 128), lambda i: (0, TAIL_BLK)),
        ],
        out_specs=pl.BlockSpec((SLABS * ROWS_PER_SLAB, 128),
                               lambda i: (i, TAIL_BLK)),
    )
    return pl.pallas_call(
        body,
        grid_spec=grid_spec,
        out_shape=jax.ShapeDtypeStruct((B * S * ROWS_PER_SLAB, V),
                                       jnp.float32),
        input_output_aliases={0: 0},
    )(out, tail, learned)


def kernel(tokens, wte, mlp_w, mlp_b, shared_prompt, u, v):
    learned = _learned_prompt(u, v, shared_prompt, mlp_w, mlp_b)
    tokens_flat = tokens.reshape(NW, TPW // CHUNK, CHUNK).astype(jnp.int32)
    wte_main = wte[:, :MAIN_W]
    wte_tail = jnp.pad(wte[:, MAIN_W:], ((0, 0), (0, 128 - (V - MAIN_W))))
    out, tail = _sc_main(tokens_flat, wte_main, wte_tail, learned)
    out = _tc_tail(out, tail, learned)
    return out.reshape(B, S, ROWS_PER_SLAB, V)
